# baseline (speedup 1.0000x reference)
import functools

import jax
import jax.numpy as jnp
from jax import lax
from jax.experimental import pallas as pl
from jax.experimental.pallas import tpu as pltpu
from jax.experimental.pallas import tpu_sc as plsc


def _sc_gather(ids_flat, emb_table):
    n = ids_flat.shape[0]
    d = emb_table.shape[1]
    info = plsc.get_sparse_core_info()
    nw = info.num_cores * info.num_subcores
    rows_per_w = n // nw
    ch = 16
    n_ch = rows_per_w // ch
    mesh = plsc.VectorSubcoreMesh(core_axis_name="c", subcore_axis_name="s")

    @functools.partial(
        pl.kernel,
        mesh=mesh,
        out_type=jax.ShapeDtypeStruct((n, d), jnp.float32),
        scratch_types=[
            pltpu.VMEM((rows_per_w,), jnp.int32),
            pltpu.VMEM((ch, d), jnp.float32),
            pltpu.SemaphoreType.DMA,
        ],
    )
    def gather_k(table_hbm, idx_hbm, out_hbm, idx_v, buf, sem):
        wid = lax.axis_index("s") * info.num_cores + lax.axis_index("c")
        base = wid * rows_per_w
        pltpu.sync_copy(idx_hbm.at[pl.ds(base, rows_per_w)], idx_v)

        def body(c, carry):
            pltpu.async_copy(
                table_hbm.at[idx_v.at[pl.ds(c * ch, ch)]], buf, sem
            ).wait()
            pltpu.sync_copy(buf, out_hbm.at[pl.ds(base + c * ch, ch)])
            return carry

        lax.fori_loop(0, n_ch, body, 0)

    return gather_k(emb_table, ids_flat)


def _tc_mlp(x, W1, b1, W2, b2):
    n, d = x.shape
    bm = 512

    def mlp_body(x_ref, w1_ref, b1_ref, w2_ref, b2_ref, o_ref):
        xv = x_ref[...]
        h = jnp.dot(xv, w1_ref[...], preferred_element_type=jnp.float32)
        h = h + b1_ref[...]
        g = 0.5 * h * (1.0 + lax.erf(h * 0.7071067811865476))
        o = jnp.dot(g, w2_ref[...], preferred_element_type=jnp.float32)
        o_ref[...] = xv + o + b2_ref[...]

    return pl.pallas_call(
        mlp_body,
        grid=(n // bm,),
        in_specs=[
            pl.BlockSpec((bm, d), lambda i: (i, 0)),
            pl.BlockSpec((d, d), lambda i: (0, 0)),
            pl.BlockSpec((1, d), lambda i: (0, 0)),
            pl.BlockSpec((d, d), lambda i: (0, 0)),
            pl.BlockSpec((1, d), lambda i: (0, 0)),
        ],
        out_specs=pl.BlockSpec((bm, d), lambda i: (i, 0)),
        out_shape=jax.ShapeDtypeStruct((n, d), jnp.float32),
    )(x, W1, b1.reshape(1, d), W2, b2.reshape(1, d))


def kernel(input_ids, emb_table, W1, b1, W2, b2):
    b, s = input_ids.shape
    d = emb_table.shape[1]
    n = b * s
    ids = input_ids.reshape(n)
    nc = 4
    cn = n // nc
    outs = []
    for c in range(nc):
        g = _sc_gather(lax.slice(ids, (c * cn,), ((c + 1) * cn,)), emb_table)
        outs.append(_tc_mlp(g, W1, b1, W2, b2))
    out = jnp.concatenate(outs, axis=0)
    return out.reshape(b, s, d)


# speedup vs baseline: 1.2419x; 1.2419x over previous
import functools

import jax
import jax.numpy as jnp
from jax import lax
from jax.experimental import pallas as pl
from jax.experimental.pallas import tpu as pltpu
from jax.experimental.pallas import tpu_sc as plsc


def _sc_gather(ids_chunk, emb_table):
    n = ids_chunk.shape[0]
    d = emb_table.shape[1]
    info = plsc.get_sparse_core_info()
    nw = info.num_cores * info.num_subcores
    rows_per_w = n // nw
    ch = 16
    n_ch = rows_per_w // ch
    mesh = plsc.VectorSubcoreMesh(core_axis_name="c", subcore_axis_name="s")

    @functools.partial(
        pl.kernel,
        mesh=mesh,
        out_type=jax.ShapeDtypeStruct((n, d), jnp.float32),
        scratch_types=[
            pltpu.VMEM((rows_per_w,), jnp.int32),
            pltpu.VMEM((ch, d), jnp.float32),
            pltpu.SemaphoreType.DMA,
        ],
    )
    def gather_k(table_hbm, idx_hbm, out_hbm, idx_v, buf, sem):
        wid = lax.axis_index("s") * info.num_cores + lax.axis_index("c")
        base = wid * rows_per_w
        pltpu.sync_copy(idx_hbm.at[pl.ds(base, rows_per_w)], idx_v)

        def body(c, carry):
            pltpu.async_copy(
                table_hbm.at[idx_v.at[pl.ds(c * ch, ch)]], buf, sem
            ).wait()
            pltpu.sync_copy(buf, out_hbm.at[pl.ds(base + c * ch, ch)])
            return carry

        lax.fori_loop(0, n_ch, body, 0)

    return gather_k(emb_table, ids_chunk)


def _tc_mlp_chunk(acc, x, W1, b1, W2, b2, n, chunk_blocks, block0):
    cn, d = x.shape
    bm = cn // chunk_blocks

    def compute(x_ref, w1_ref, b1_ref, w2_ref, b2_ref, o_ref):
        xv = x_ref[...]
        h = jnp.dot(xv, w1_ref[...], preferred_element_type=jnp.float32)
        h = h + b1_ref[...]
        g = 0.5 * h * (1.0 + lax.erf(h * 0.7071067811865476))
        o = jnp.dot(g, w2_ref[...], preferred_element_type=jnp.float32)
        o_ref[...] = xv + o + b2_ref[...]

    data_specs = [
        pl.BlockSpec((bm, d), lambda i: (i, 0)),
        pl.BlockSpec((d, d), lambda i: (0, 0)),
        pl.BlockSpec((1, d), lambda i: (0, 0)),
        pl.BlockSpec((d, d), lambda i: (0, 0)),
        pl.BlockSpec((1, d), lambda i: (0, 0)),
    ]
    data_args = (x, W1, b1.reshape(1, d), W2, b2.reshape(1, d))

    if acc is None:
        body = compute
        in_specs = data_specs
        args = data_args
        aliases = {}
    else:
        def body(acc_ref, *rest):
            del acc_ref
            compute(*rest)

        in_specs = [pl.BlockSpec(memory_space=pl.ANY)] + data_specs
        args = (acc,) + data_args
        aliases = {0: 0}

    return pl.pallas_call(
        body,
        grid=(chunk_blocks,),
        in_specs=in_specs,
        out_specs=pl.BlockSpec((bm, d), lambda i: (block0 + i, 0)),
        out_shape=jax.ShapeDtypeStruct((n, d), jnp.float32),
        input_output_aliases=aliases,
    )(*args)


def kernel(input_ids, emb_table, W1, b1, W2, b2):
    b, s = input_ids.shape
    d = emb_table.shape[1]
    n = b * s
    ids = input_ids.reshape(n)
    nc = 4
    chunk_blocks = 8
    cn = n // nc
    acc = None
    for c in range(nc):
        g = _sc_gather(lax.slice(ids, (c * cn,), ((c + 1) * cn,)), emb_table)
        acc = _tc_mlp_chunk(
            acc, g, W1, b1, W2, b2, n, chunk_blocks, c * chunk_blocks
        )
    return acc.reshape(b, s, d)


# speedup vs baseline: 1.2900x; 1.0387x over previous
import functools

import jax
import jax.numpy as jnp
from jax import lax
from jax.experimental import pallas as pl
from jax.experimental.pallas import tpu as pltpu
from jax.experimental.pallas import tpu_sc as plsc


def _sc_gather(ids_chunk, emb_table):
    n = ids_chunk.shape[0]
    d = emb_table.shape[1]
    info = plsc.get_sparse_core_info()
    nw = info.num_cores * info.num_subcores
    rows_per_w = n // nw
    ch = 16
    n_ch = rows_per_w // ch
    mesh = plsc.VectorSubcoreMesh(core_axis_name="c", subcore_axis_name="s")

    @functools.partial(
        pl.kernel,
        mesh=mesh,
        out_type=jax.ShapeDtypeStruct((n, d), jnp.float32),
        scratch_types=[
            pltpu.VMEM((rows_per_w,), jnp.int32),
            pltpu.VMEM((ch, d), jnp.float32),
            pltpu.SemaphoreType.DMA,
        ],
    )
    def gather_k(table_hbm, idx_hbm, out_hbm, idx_v, buf, sem):
        wid = lax.axis_index("s") * info.num_cores + lax.axis_index("c")
        base = wid * rows_per_w
        pltpu.sync_copy(idx_hbm.at[pl.ds(base, rows_per_w)], idx_v)

        def body(c, carry):
            pltpu.async_copy(
                table_hbm.at[idx_v.at[pl.ds(c * ch, ch)]], buf, sem
            ).wait()
            pltpu.sync_copy(buf, out_hbm.at[pl.ds(base + c * ch, ch)])
            return carry

        lax.fori_loop(0, n_ch, body, 0)

    return gather_k(emb_table, ids_chunk)


def _tc_mlp_chunk(acc, x, W1, b1, W2, b2, n, chunk_blocks, block0):
    cn, d = x.shape
    bm = cn // chunk_blocks

    def compute(x_ref, w1_ref, b1_ref, w2_ref, b2_ref, o_ref):
        xv = x_ref[...]
        h = jnp.dot(xv, w1_ref[...], preferred_element_type=jnp.float32)
        h = h + b1_ref[...]
        g = 0.5 * h * (1.0 + lax.erf(h * 0.7071067811865476))
        o = jnp.dot(g, w2_ref[...], preferred_element_type=jnp.float32)
        o_ref[...] = xv + o + b2_ref[...]

    data_specs = [
        pl.BlockSpec((bm, d), lambda i: (i, 0)),
        pl.BlockSpec((d, d), lambda i: (0, 0)),
        pl.BlockSpec((1, d), lambda i: (0, 0)),
        pl.BlockSpec((d, d), lambda i: (0, 0)),
        pl.BlockSpec((1, d), lambda i: (0, 0)),
    ]
    data_args = (x, W1, b1.reshape(1, d), W2, b2.reshape(1, d))

    if acc is None:
        body = compute
        in_specs = data_specs
        args = data_args
        aliases = {}
    else:
        def body(acc_ref, *rest):
            del acc_ref
            compute(*rest)

        in_specs = [pl.BlockSpec(memory_space=pl.ANY)] + data_specs
        args = (acc,) + data_args
        aliases = {0: 0}

    return pl.pallas_call(
        body,
        grid=(chunk_blocks,),
        in_specs=in_specs,
        out_specs=pl.BlockSpec((bm, d), lambda i: (block0 + i, 0)),
        out_shape=jax.ShapeDtypeStruct((n, d), jnp.float32),
        input_output_aliases=aliases,
    )(*args)


def kernel(input_ids, emb_table, W1, b1, W2, b2):
    b, s = input_ids.shape
    d = emb_table.shape[1]
    n = b * s
    ids = input_ids.reshape(n)
    nc = 2
    chunk_blocks = 16
    cn = n // nc
    acc = None
    for c in range(nc):
        g = _sc_gather(lax.slice(ids, (c * cn,), ((c + 1) * cn,)), emb_table)
        acc = _tc_mlp_chunk(
            acc, g, W1, b1, W2, b2, n, chunk_blocks, c * chunk_blocks
        )
    return acc.reshape(b, s, d)


# speedup vs baseline: 1.3234x; 1.0259x over previous
import functools

import jax
import jax.numpy as jnp
from jax import lax
from jax.experimental import pallas as pl
from jax.experimental.pallas import tpu as pltpu
from jax.experimental.pallas import tpu_sc as plsc


def _sc_gather(ids_chunk, emb_table):
    n = ids_chunk.shape[0]
    d = emb_table.shape[1]
    info = plsc.get_sparse_core_info()
    nw = info.num_cores * info.num_subcores
    rows_per_w = n // nw
    ch = 16
    n_ch = rows_per_w // ch
    mesh = plsc.VectorSubcoreMesh(core_axis_name="c", subcore_axis_name="s")

    @functools.partial(
        pl.kernel,
        mesh=mesh,
        out_type=jax.ShapeDtypeStruct((n, d), jnp.float32),
        scratch_types=[
            pltpu.VMEM((rows_per_w,), jnp.int32),
            pltpu.VMEM((ch, d), jnp.float32),
            pltpu.SemaphoreType.DMA,
        ],
    )
    def gather_k(table_hbm, idx_hbm, out_hbm, idx_v, buf, sem):
        wid = lax.axis_index("s") * info.num_cores + lax.axis_index("c")
        base = wid * rows_per_w
        pltpu.sync_copy(idx_hbm.at[pl.ds(base, rows_per_w)], idx_v)

        def body(c, carry):
            pltpu.async_copy(
                table_hbm.at[idx_v.at[pl.ds(c * ch, ch)]], buf, sem
            ).wait()
            pltpu.sync_copy(buf, out_hbm.at[pl.ds(base + c * ch, ch)])
            return carry

        lax.fori_loop(0, n_ch, body, 0)

    return gather_k(emb_table, ids_chunk)


def _tc_mlp_chunk(acc, x, W1, b1, W2, b2, n, chunk_blocks, block0):
    cn, d = x.shape
    bm = cn // chunk_blocks

    def compute(x_ref, w1_ref, b1_ref, w2_ref, b2_ref, o_ref):
        xv = x_ref[...]
        h = jnp.dot(xv, w1_ref[...], preferred_element_type=jnp.float32)
        h = h + b1_ref[...]
        g = 0.5 * h * (1.0 + lax.erf(h * 0.7071067811865476))
        o = jnp.dot(g, w2_ref[...], preferred_element_type=jnp.float32)
        o_ref[...] = xv + o + b2_ref[...]

    data_specs = [
        pl.BlockSpec((bm, d), lambda i: (i, 0)),
        pl.BlockSpec((d, d), lambda i: (0, 0)),
        pl.BlockSpec((1, d), lambda i: (0, 0)),
        pl.BlockSpec((d, d), lambda i: (0, 0)),
        pl.BlockSpec((1, d), lambda i: (0, 0)),
    ]
    data_args = (x, W1, b1.reshape(1, d), W2, b2.reshape(1, d))

    if acc is None:
        body = compute
        in_specs = data_specs
        args = data_args
        aliases = {}
    else:
        def body(acc_ref, *rest):
            del acc_ref
            compute(*rest)

        in_specs = [pl.BlockSpec(memory_space=pl.ANY)] + data_specs
        args = (acc,) + data_args
        aliases = {0: 0}

    return pl.pallas_call(
        body,
        grid=(chunk_blocks,),
        in_specs=in_specs,
        out_specs=pl.BlockSpec((bm, d), lambda i: (block0 + i, 0)),
        out_shape=jax.ShapeDtypeStruct((n, d), jnp.float32),
        input_output_aliases=aliases,
    )(*args)


def kernel(input_ids, emb_table, W1, b1, W2, b2):
    b, s = input_ids.shape
    d = emb_table.shape[1]
    n = b * s
    bm = 512
    ids = input_ids.reshape(n)
    chunk_plan = [8, 24]
    acc = None
    block0 = 0
    for nblk in chunk_plan:
        cn = nblk * bm
        g = _sc_gather(
            lax.slice(ids, (block0 * bm,), (block0 * bm + cn,)), emb_table
        )
        acc = _tc_mlp_chunk(acc, g, W1, b1, W2, b2, n, nblk, block0)
        block0 += nblk
    return acc.reshape(b, s, d)


# speedup vs baseline: 1.3772x; 1.0406x over previous
import functools

import jax
import jax.numpy as jnp
from jax import lax
from jax.experimental import pallas as pl
from jax.experimental.pallas import tpu as pltpu
from jax.experimental.pallas import tpu_sc as plsc


def _sc_gather(ids_chunk, emb_table):
    n = ids_chunk.shape[0]
    d = emb_table.shape[1]
    info = plsc.get_sparse_core_info()
    nw = info.num_cores * info.num_subcores
    rows_per_w = n // nw
    ch = 16
    n_ch = rows_per_w // ch
    mesh = plsc.VectorSubcoreMesh(core_axis_name="c", subcore_axis_name="s")

    @functools.partial(
        pl.kernel,
        mesh=mesh,
        out_type=jax.ShapeDtypeStruct((n, d), jnp.float32),
        scratch_types=[
            pltpu.VMEM((rows_per_w,), jnp.int32),
            pltpu.VMEM((ch, d), jnp.float32),
            pltpu.VMEM((ch, d), jnp.float32),
            pltpu.SemaphoreType.DMA,
            pltpu.SemaphoreType.DMA,
        ],
    )
    def gather_k(table_hbm, idx_hbm, out_hbm, idx_v, buf0, buf1, sem0, sem1):
        wid = lax.axis_index("s") * info.num_cores + lax.axis_index("c")
        base = wid * rows_per_w
        pltpu.sync_copy(idx_hbm.at[pl.ds(base, rows_per_w)], idx_v)

        def start(c, buf, sem):
            pltpu.async_copy(table_hbm.at[idx_v.at[pl.ds(c * ch, ch)]], buf, sem)

        def wait(buf, sem):
            pltpu.make_async_copy(
                table_hbm.at[idx_v.at[pl.ds(0, ch)]], buf, sem
            ).wait()

        start(0, buf0, sem0)
        start(1, buf1, sem1)

        def body(p, carry):
            c0 = 2 * p
            wait(buf0, sem0)
            pltpu.sync_copy(buf0, out_hbm.at[pl.ds(base + c0 * ch, ch)])

            @pl.when(c0 + 2 < n_ch)
            def _():
                start(c0 + 2, buf0, sem0)

            wait(buf1, sem1)
            pltpu.sync_copy(buf1, out_hbm.at[pl.ds(base + (c0 + 1) * ch, ch)])

            @pl.when(c0 + 3 < n_ch)
            def _():
                start(c0 + 3, buf1, sem1)

            return carry

        lax.fori_loop(0, n_ch // 2, body, 0)

    return gather_k(emb_table, ids_chunk)


def _tc_mlp_chunk(acc, x, W1, b1, W2, b2, n, chunk_blocks, block0):
    cn, d = x.shape
    bm = cn // chunk_blocks

    def compute(x_ref, w1_ref, b1_ref, w2_ref, b2_ref, o_ref):
        xv = x_ref[...]
        h = jnp.dot(xv, w1_ref[...], preferred_element_type=jnp.float32)
        h = h + b1_ref[...]
        g = 0.5 * h * (1.0 + lax.erf(h * 0.7071067811865476))
        o = jnp.dot(g, w2_ref[...], preferred_element_type=jnp.float32)
        o_ref[...] = xv + o + b2_ref[...]

    data_specs = [
        pl.BlockSpec((bm, d), lambda i: (i, 0)),
        pl.BlockSpec((d, d), lambda i: (0, 0)),
        pl.BlockSpec((1, d), lambda i: (0, 0)),
        pl.BlockSpec((d, d), lambda i: (0, 0)),
        pl.BlockSpec((1, d), lambda i: (0, 0)),
    ]
    data_args = (x, W1, b1.reshape(1, d), W2, b2.reshape(1, d))

    if acc is None:
        body = compute
        in_specs = data_specs
        args = data_args
        aliases = {}
    else:
        def body(acc_ref, *rest):
            del acc_ref
            compute(*rest)

        in_specs = [pl.BlockSpec(memory_space=pl.ANY)] + data_specs
        args = (acc,) + data_args
        aliases = {0: 0}

    return pl.pallas_call(
        body,
        grid=(chunk_blocks,),
        in_specs=in_specs,
        out_specs=pl.BlockSpec((bm, d), lambda i: (block0 + i, 0)),
        out_shape=jax.ShapeDtypeStruct((n, d), jnp.float32),
        input_output_aliases=aliases,
    )(*args)


def kernel(input_ids, emb_table, W1, b1, W2, b2):
    b, s = input_ids.shape
    d = emb_table.shape[1]
    n = b * s
    bm = 512
    ids = input_ids.reshape(n)
    chunk_plan = [8, 24]
    acc = None
    block0 = 0
    for nblk in chunk_plan:
        cn = nblk * bm
        g = _sc_gather(
            lax.slice(ids, (block0 * bm,), (block0 * bm + cn,)), emb_table
        )
        acc = _tc_mlp_chunk(acc, g, W1, b1, W2, b2, n, nblk, block0)
        block0 += nblk
    return acc.reshape(b, s, d)
